# baseline (speedup 1.0000x reference)
import jax
import jax.numpy as jnp
from jax.experimental import pallas as pl
from jax.experimental.pallas import tpu as pltpu

K_Q = 65536
D = 128
B_K = 4096
R = 512


def _body(ptr_s, kpad_ref, queue_ref, out_ref):
    i = pl.program_id(0)
    r0 = i * R
    p = ptr_s[0]
    i0 = (r0 - p) & (K_Q - 1)
    s = jnp.where(i0 < B_K, R + i0, jnp.maximum(0, i0 - (K_Q - R)))
    src = kpad_ref[pl.ds(s, R), :]
    rows = jax.lax.broadcasted_iota(jnp.int32, (R, D), 0) + r0
    in_win = ((rows - p) & (K_Q - 1)) < B_K
    out_ref[...] = jnp.where(in_win, src, queue_ref[...])


def kernel(keys, queue, ptr):
    pad = jnp.zeros((R, D), dtype=keys.dtype)
    kpad = jnp.concatenate([pad, keys, pad], axis=0)
    grid_spec = pltpu.PrefetchScalarGridSpec(
        num_scalar_prefetch=1,
        grid=(K_Q // R,),
        in_specs=[
            pl.BlockSpec((B_K + 2 * R, D), lambda i, p: (0, 0)),
            pl.BlockSpec((R, D), lambda i, p: (i, 0)),
        ],
        out_specs=pl.BlockSpec((R, D), lambda i, p: (i, 0)),
    )
    return pl.pallas_call(
        _body,
        grid_spec=grid_spec,
        out_shape=jax.ShapeDtypeStruct((K_Q, D), queue.dtype),
        compiler_params=pltpu.CompilerParams(
            dimension_semantics=("arbitrary",),
        ),
    )(ptr.astype(jnp.int32), kpad, queue)


# speedup vs baseline: 2.6093x; 2.6093x over previous
import jax
import jax.numpy as jnp
from jax.experimental import pallas as pl
from jax.experimental.pallas import tpu as pltpu

K_Q = 65536
D = 128
B_K = 4096
R = 512
NBLK = K_Q // R
NWIN = B_K // R + 1


def _blk(i, p):
    return (p[0] // R + i) % NBLK, 0


def _body(ptr_s, kpad_ref, queue_ref, out_ref):
    i = pl.program_id(0)
    p = ptr_s[0]
    r0 = ((p // R + i) % NBLK) * R
    i0 = (r0 - p) & (K_Q - 1)
    s = jnp.where(i0 < B_K, R + i0, jnp.maximum(0, i0 - (K_Q - R)))
    src = kpad_ref[pl.ds(s, R), :]
    rows = jax.lax.broadcasted_iota(jnp.int32, (R, D), 0) + r0
    in_win = ((rows - p) & (K_Q - 1)) < B_K
    out_ref[...] = jnp.where(in_win, src, queue_ref[...])


def kernel(keys, queue, ptr):
    pad = jnp.zeros((R, D), dtype=keys.dtype)
    kpad = jnp.concatenate([pad, keys, pad], axis=0)
    grid_spec = pltpu.PrefetchScalarGridSpec(
        num_scalar_prefetch=1,
        grid=(NWIN,),
        in_specs=[
            pl.BlockSpec((B_K + 2 * R, D), lambda i, p: (0, 0)),
            pl.BlockSpec((R, D), _blk),
        ],
        out_specs=pl.BlockSpec((R, D), _blk),
    )
    return pl.pallas_call(
        _body,
        grid_spec=grid_spec,
        out_shape=jax.ShapeDtypeStruct((K_Q, D), queue.dtype),
        input_output_aliases={2: 0},
        compiler_params=pltpu.CompilerParams(
            dimension_semantics=("arbitrary",),
        ),
    )(ptr.astype(jnp.int32), kpad, queue)
